# baseline (speedup 1.0000x reference)
import jax
import jax.numpy as jnp
from jax import lax
from jax.experimental import pallas as pl
from jax.experimental.pallas import tpu as pltpu


def _body(x_ref, out_ref, *, scale):
    i = pl.program_id(0)
    j = pl.program_id(1)

    @pl.when(jnp.logical_and(i == 0, j == 0))
    def _():
        out_ref[0, 0] = 0.0

    x = x_ref[...]
    s = jnp.sum(x * x, axis=2, keepdims=True)
    y = x * lax.rsqrt(jnp.maximum(s, 1e-24))
    d = jnp.abs(y[:-1] - y[1:])
    out_ref[0, 0] += jnp.sum(d) * scale


def kernel(feats, scores, masks):
    del scores, masks
    F, n, c, h, w = feats.shape
    hb = next(x for x in (128, 64, 32, 16, 8, 4, 2, 1) if h % x == 0)
    n_h = h // hb
    scale = 1.0 / ((F - 1) * n * c * h * w)

    out = pl.pallas_call(
        lambda x_ref, out_ref: _body(x_ref, out_ref, scale=scale),
        grid=(n, n_h),
        in_specs=[
            pl.BlockSpec((F, 1, c, hb, w), lambda i, j: (0, i, 0, j, 0)),
        ],
        out_specs=pl.BlockSpec(
            (1, 1), lambda i, j: (0, 0), memory_space=pltpu.SMEM
        ),
        out_shape=jax.ShapeDtypeStruct((1, 1), jnp.float32),
        compiler_params=pltpu.CompilerParams(
            vmem_limit_bytes=100 * 1024 * 1024
        ),
    )(feats)
    return out[0, 0]


# speedup vs baseline: 1.0249x; 1.0249x over previous
import jax
import jax.numpy as jnp
from jax import lax
from jax.experimental import pallas as pl
from jax.experimental.pallas import tpu as pltpu


def _body(x_ref, out_ref, *, scale):
    i = pl.program_id(0)
    j = pl.program_id(1)

    @pl.when(jnp.logical_and(i == 0, j == 0))
    def _():
        out_ref[0, 0] = 0.0

    x = x_ref[...]
    sumsq = jnp.sum(x * x, axis=2, keepdims=True)
    y = x * lax.rsqrt(jnp.maximum(sumsq, 1e-24))
    out_ref[0, 0] += jnp.sum(jnp.abs(y[1:] - y[:-1])) * scale


def kernel(feats, scores, masks):
    del scores, masks
    F, n, c, h, w = feats.shape
    hb = next(x for x in (64, 32, 16, 8, 4, 2, 1) if h % x == 0)
    n_h = h // hb
    scale = 1.0 / ((F - 1) * n * c * h * w)

    out = pl.pallas_call(
        lambda x_ref, out_ref: _body(x_ref, out_ref, scale=scale),
        grid=(n, n_h),
        in_specs=[
            pl.BlockSpec((F, 1, c, hb, w), lambda i, j: (0, i, 0, j, 0)),
        ],
        out_specs=pl.BlockSpec(
            (1, 1), lambda i, j: (0, 0), memory_space=pltpu.SMEM
        ),
        out_shape=jax.ShapeDtypeStruct((1, 1), jnp.float32),
    )(feats)
    return out[0, 0]
